# Initial kernel scaffold; baseline (speedup 1.0000x reference)
import jax
import jax.numpy as jnp
from jax.experimental import pallas as pl


def kernel(hidden_states, attention_weights, W_full, W_sp1, b_sp1, W_sp2, b_sp2, W_min, b_min, W_exp, b_exp):
    raise NotImplementedError("write your pallas kernel here")


# speedup vs baseline: 1.5764x; 1.5764x over previous
import jax
import jax.numpy as jnp
from jax.experimental import pallas as pl
from jax.experimental.pallas import tpu as pltpu

MODEL_DIM = 768
VOCAB = 16000
F_SP = 4000
F_MIN = 2000
HIGH_T = 0.7
MED_T = 0.3
S = 2048

VT = 384
N_VT = (VOCAB + VT - 1) // VT
F_SP_T = 1024
F_MIN_T = 512
N_FT = 4


def _strategy_body(att_ref, x_ref, strat_ref, xbf_ref):
    ta = jnp.sum(att_ref[...], axis=1, keepdims=True)
    mx = jnp.max(ta)
    norm = ta / (mx + 1e-8)
    strat = jnp.where(norm >= HIGH_T, 2, jnp.where(norm >= MED_T, 1, 0))
    strat_ref[...] = jnp.broadcast_to(strat.astype(jnp.int32), (S, 128))
    xbf_ref[...] = x_ref[...].astype(jnp.bfloat16)


def _stage1_body(x_ref, wsp1_ref, bsp1_ref, wmin_ref, bmin_ref, h1_ref, mf_ref):
    xb = x_ref[...].astype(jnp.bfloat16)
    a1 = jnp.dot(xb, wsp1_ref[...].astype(jnp.bfloat16),
                 preferred_element_type=jnp.float32) + bsp1_ref[...]
    h1 = 0.5 * a1 * (1.0 + jax.lax.erf(a1 / jnp.sqrt(2.0).astype(jnp.float32)))
    h1_ref[...] = h1.astype(jnp.bfloat16)
    a0 = jnp.dot(xb, wmin_ref[...].astype(jnp.bfloat16),
                 preferred_element_type=jnp.float32) + bmin_ref[...]
    mf_ref[...] = a0.astype(jnp.bfloat16)


def _proj_body(strat_ref, xbf_ref, h1_ref, mf_ref, wf_ref, ws2_ref, bs2_ref,
               we_ref, be_ref, out_ref):
    wf = wf_ref[...].astype(jnp.bfloat16)
    ws2 = ws2_ref[...].astype(jnp.bfloat16)
    we = we_ref[...].astype(jnp.bfloat16)
    bs2 = bs2_ref[...]
    be = be_ref[...]
    RT = 512
    for i in range(S // RT):
        rs = pl.ds(i * RT, RT)
        s = strat_ref[rs, :][:, :1]
        mn = jnp.dot(mf_ref[rs, :], we, preferred_element_type=jnp.float32) + be
        sp = jnp.dot(h1_ref[rs, :], ws2, preferred_element_type=jnp.float32) + bs2
        r = jnp.where(s == 1, sp, mn)
        fl = jnp.dot(xbf_ref[rs, :], wf, preferred_element_type=jnp.float32)
        out_ref[rs, :] = jnp.where(s == 2, fl, r)


def kernel(hidden_states, attention_weights, W_full, W_sp1, b_sp1, W_sp2, b_sp2,
           W_min, b_min, W_exp, b_exp):
    x = hidden_states.reshape(S, MODEL_DIM)
    att = attention_weights.reshape(S, -1)

    strat, xbf = pl.pallas_call(
        _strategy_body,
        out_shape=[
            jax.ShapeDtypeStruct((S, 128), jnp.int32),
            jax.ShapeDtypeStruct((S, MODEL_DIM), jnp.bfloat16),
        ],
    )(att, x)

    h1, mf = pl.pallas_call(
        _stage1_body,
        grid=(N_FT,),
        in_specs=[
            pl.BlockSpec((S, MODEL_DIM), lambda j: (0, 0)),
            pl.BlockSpec((MODEL_DIM, F_SP_T), lambda j: (0, j)),
            pl.BlockSpec((1, F_SP_T), lambda j: (0, j)),
            pl.BlockSpec((MODEL_DIM, F_MIN_T), lambda j: (0, j)),
            pl.BlockSpec((1, F_MIN_T), lambda j: (0, j)),
        ],
        out_specs=[
            pl.BlockSpec((S, F_SP_T), lambda j: (0, j)),
            pl.BlockSpec((S, F_MIN_T), lambda j: (0, j)),
        ],
        out_shape=[
            jax.ShapeDtypeStruct((S, F_SP), jnp.bfloat16),
            jax.ShapeDtypeStruct((S, F_MIN), jnp.bfloat16),
        ],
    )(x, W_sp1, b_sp1.reshape(1, -1), W_min, b_min.reshape(1, -1))

    out = pl.pallas_call(
        _proj_body,
        grid=(N_VT,),
        in_specs=[
            pl.BlockSpec((S, 128), lambda j: (0, 0)),
            pl.BlockSpec((S, MODEL_DIM), lambda j: (0, 0)),
            pl.BlockSpec((S, F_SP), lambda j: (0, 0)),
            pl.BlockSpec((S, F_MIN), lambda j: (0, 0)),
            pl.BlockSpec((MODEL_DIM, VT), lambda j: (0, j)),
            pl.BlockSpec((F_SP, VT), lambda j: (0, j)),
            pl.BlockSpec((1, VT), lambda j: (0, j)),
            pl.BlockSpec((F_MIN, VT), lambda j: (0, j)),
            pl.BlockSpec((1, VT), lambda j: (0, j)),
        ],
        out_specs=pl.BlockSpec((S, VT), lambda j: (0, j)),
        out_shape=jax.ShapeDtypeStruct((S, VOCAB), jnp.float32),
        compiler_params=pltpu.CompilerParams(
            vmem_limit_bytes=64 * 1024 * 1024),
    )(strat, xbf, h1, mf, W_full, W_sp2, b_sp2.reshape(1, -1),
      W_exp, b_exp.reshape(1, -1))

    return out.reshape(1, S, VOCAB)
